# baseline (speedup 1.0000x reference)
import jax
import jax.numpy as jnp
from jax.experimental import pallas as pl
from jax.experimental.pallas import tpu as pltpu

_B = 8
_N = 4096
_D = 512
_C = 1024
_NU = 0.001
_BM = 4096
_STEPS = _B * _N // _BM


def _body(e_ref, ctb_ref, cen_ref, r_ref, mins_ref, part_ref, eout_ref):
    e = e_ref[...]
    eout_ref[...] = e
    g = jnp.dot(
        (-2.0 * e).astype(jnp.bfloat16),
        ctb_ref[...],
        preferred_element_type=jnp.float32,
    )
    feat = jnp.sum(e * e, axis=1, keepdims=True)
    d2 = cen_ref[...] + g
    mind = jnp.sqrt(jnp.min(d2, axis=1, keepdims=True) + feat)
    mins_ref[...] = mind
    r2 = r_ref[0] * r_ref[0]
    part_ref[0, 0, 0] = jnp.sum(jnp.maximum(mind - r2, 0.0))


def kernel(embeds, centroids, r):
    e2 = embeds.reshape(_B * _N, _D)
    ctb = centroids.T.astype(jnp.bfloat16)
    cen = jnp.sum(centroids * centroids, axis=1)[None, :]
    mins, parts, eout = pl.pallas_call(
        _body,
        grid=(_STEPS,),
        in_specs=[
            pl.BlockSpec((_BM, _D), lambda i: (i, 0)),
            pl.BlockSpec((_D, _C), lambda i: (0, 0)),
            pl.BlockSpec((1, _C), lambda i: (0, 0)),
            pl.BlockSpec(memory_space=pltpu.SMEM),
        ],
        out_specs=[
            pl.BlockSpec((_BM, 1), lambda i: (i, 0)),
            pl.BlockSpec((1, 1, 1), lambda i: (i, 0, 0),
                         memory_space=pltpu.SMEM),
            pl.BlockSpec((_BM, _D), lambda i: (i, 0)),
        ],
        out_shape=[
            jax.ShapeDtypeStruct((_B * _N, 1), jnp.float32),
            jax.ShapeDtypeStruct((_STEPS, 1, 1), jnp.float32),
            jax.ShapeDtypeStruct((_B * _N, _D), jnp.float32),
        ],
        compiler_params=pltpu.CompilerParams(
            dimension_semantics=("parallel",),
        ),
    )(e2, ctb, cen, r)
    h = 64
    score = mins.reshape(_B, 1, h, h)
    loss = jnp.sum(parts) / (_NU * _B * _N)
    return (loss, score, eout.reshape(_B, _N, _D))


# speedup vs baseline: 1.0405x; 1.0405x over previous
import jax
import jax.numpy as jnp
from jax.experimental import pallas as pl
from jax.experimental.pallas import tpu as pltpu

_B = 8
_N = 4096
_D = 512
_C = 1024
_NU = 0.001
_BM = 4096


def _body(e_ref, ct_ref, r_ref, mins_ref, loss_ref, eout_ref, ctb_ref, cen_ref):
    i = pl.program_id(0)

    @pl.when(i == 0)
    def _prep():
        ct = ct_ref[...]
        ctb_ref[...] = ct.astype(jnp.bfloat16)
        cen_ref[...] = jnp.sum(ct * ct, axis=0, keepdims=True)

    e = e_ref[...]
    eout_ref[...] = e
    g = jnp.dot(
        (-2.0 * e).astype(jnp.bfloat16),
        ctb_ref[...],
        preferred_element_type=jnp.float32,
    )
    feat = jnp.sum(e * e, axis=1, keepdims=True)
    d2 = cen_ref[...] + g
    mind = jnp.sqrt(jnp.min(d2, axis=1, keepdims=True) + feat)
    mins_ref[...] = mind
    r2 = r_ref[0] * r_ref[0]
    bs = jnp.sum(jnp.maximum(mind - r2, 0.0))

    @pl.when(i == 0)
    def _init():
        loss_ref[0, 0] = 0.0

    loss_ref[0, 0] += bs


def kernel(embeds, centroids, r):
    e2 = embeds.reshape(_B * _N, _D)
    ct = centroids.T
    mins, losssum, eout = pl.pallas_call(
        _body,
        grid=(_B * _N // _BM,),
        in_specs=[
            pl.BlockSpec((_BM, _D), lambda i: (i, 0)),
            pl.BlockSpec((_D, _C), lambda i: (0, 0)),
            pl.BlockSpec(memory_space=pltpu.SMEM),
        ],
        out_specs=[
            pl.BlockSpec((_BM, 1), lambda i: (i, 0)),
            pl.BlockSpec(memory_space=pltpu.SMEM),
            pl.BlockSpec((_BM, _D), lambda i: (i, 0)),
        ],
        out_shape=[
            jax.ShapeDtypeStruct((_B * _N, 1), jnp.float32),
            jax.ShapeDtypeStruct((1, 1), jnp.float32),
            jax.ShapeDtypeStruct((_B * _N, _D), jnp.float32),
        ],
        scratch_shapes=[
            pltpu.VMEM((_D, _C), jnp.bfloat16),
            pltpu.VMEM((1, _C), jnp.float32),
        ],
    )(e2, ct, r)
    h = 64
    score = mins.reshape(_B, 1, h, h)
    loss = losssum[0, 0] / (_NU * _B * _N)
    return (loss, score, eout.reshape(_B, _N, _D))
